# baseline (speedup 1.0000x reference)
import functools
import math

import jax
import jax.numpy as jnp
from jax import lax
from jax.experimental import pallas as pl
from jax.experimental.pallas import tpu as pltpu
from jax.experimental.pallas import tpu_sc as plsc

_LANES = 16
_CHUNK = 128
_NBUF = 4


@functools.lru_cache(maxsize=None)
def _build(vocab: int, dim: int, batch: int, hist: int):
    info = plsc.get_sparse_core_info()
    nc, ns = info.num_cores, info.num_subcores
    nw = nc * ns
    scale = math.sqrt(dim)
    per_b = batch // nw
    jn = per_b // _CHUNK
    assert jn == _NBUF and dim % _LANES == 0 and dim == 2 * _LANES * 2

    mesh = plsc.VectorSubcoreMesh(core_axis_name="c", subcore_axis_name="s")

    @functools.partial(
        pl.kernel,
        mesh=mesh,
        out_type=jax.ShapeDtypeStruct((hist, batch // _CHUNK, dim // 8, 8, _CHUNK),
                                      jnp.float32),
        scratch_types=[
            pltpu.VMEM((hist, per_b), jnp.int32),
            pltpu.VMEM((_NBUF, _CHUNK, dim), jnp.float32),
            pltpu.VMEM((_NBUF, dim // 8, 8, _CHUNK), jnp.float32),
            pltpu.SemaphoreType.DMA,
            pltpu.SemaphoreType.DMA((_NBUF,)),
            pltpu.SemaphoreType.DMA((_NBUF,)),
        ],
        compiler_params=pltpu.CompilerParams(use_tc_tiling_on_sc=False,
                                             needs_layout_passes=False),
    )
    def emb_kernel(table_hbm, idx_hbm, out_hbm, idx_all, bin_v, bout_v,
                   sem_idx, sem_in, sem_out):
        wid = lax.axis_index("s") * nc + lax.axis_index("c")
        wb0 = wid * per_b

        def fire(h, c):
            pltpu.async_copy(idx_hbm.at[pl.ds(h * batch + wb0, per_b)],
                             idx_all.at[h], sem_idx)
            return c

        lax.fori_loop(0, hist, fire, 0)

        def drain(h, c):
            pltpu.make_async_copy(idx_hbm.at[pl.ds(0, per_b)],
                                  idx_all.at[0], sem_idx).wait()
            return c

        lax.fori_loop(0, hist, drain, 0)

        def start_gather(h, j, b):
            pltpu.async_copy(
                table_hbm.at[idx_all.at[h, pl.ds(j * _CHUNK, _CHUNK)]],
                bin_v.at[b], sem_in.at[b])

        def wait_gather(b):
            pltpu.make_async_copy(
                table_hbm.at[idx_all.at[0, pl.ds(0, _CHUNK)]], bin_v.at[b],
                sem_in.at[b]).wait()

        def start_wb(h, j, b):
            pltpu.async_copy(bout_v.at[b],
                             out_hbm.at[h, wid * jn + j],
                             sem_out.at[b])

        def wait_wb(b):
            pltpu.make_async_copy(bout_v.at[b], out_hbm.at[0, 0],
                                  sem_out.at[b]).wait()

        def compute(h, j, b):
            def qloop(r, c):
                for cc in range(dim // _LANES):
                    sl = pl.ds(cc * _LANES, _LANES)
                    v = bin_v[b, r, sl]
                    bout_v[b, cc // 2, cc % 8 if cc < 8 else 0, sl] = v * scale
                return c
            lax.fori_loop(0, _CHUNK, qloop, 0)

        for b in range(_NBUF):
            start_gather(0, b, b)

        def mbody(h, c):
            for b in range(_NBUF):
                wait_gather(b)

                @pl.when(h >= 1)
                def _():
                    wait_wb(b)

                compute(h, b, b)

                @pl.when(h + 1 < hist)
                def _():
                    start_gather(h + 1, b, b)

                start_wb(h, b, b)
            return c

        lax.fori_loop(0, hist, mbody, 0)
        for b in range(_NBUF):
            wait_wb(b)

    return emb_kernel


def kernel(persona, table):
    batch, hist = persona.shape
    vocab, dim = table.shape
    idx_flat = jnp.ravel(persona.T).astype(jnp.int32)
    out5 = _build(vocab, dim, batch, hist)(table, idx_flat)
    out = jnp.transpose(out5, (1, 4, 0, 2, 3)).reshape(batch, hist, dim)
    return out


# speedup vs baseline: 1.2119x; 1.2119x over previous
import functools
import math

import jax
import jax.numpy as jnp
from jax import lax
from jax.experimental import pallas as pl
from jax.experimental.pallas import tpu as pltpu
from jax.experimental.pallas import tpu_sc as plsc

_LANES = 16
_CHUNK = 128
_NBUF = 4
_ROWS_PER_IT = 4


@functools.lru_cache(maxsize=None)
def _build(vocab: int, dim: int, batch: int, hist: int):
    info = plsc.get_sparse_core_info()
    nc, ns = info.num_cores, info.num_subcores
    nw = nc * ns
    scale = math.sqrt(dim)
    per_b = batch // nw
    jn = per_b // _CHUNK
    assert jn == _NBUF and dim % _LANES == 0

    mesh = plsc.VectorSubcoreMesh(core_axis_name="c", subcore_axis_name="s")

    @functools.partial(
        pl.kernel,
        mesh=mesh,
        out_type=jax.ShapeDtypeStruct((hist, batch, dim), jnp.float32),
        scratch_types=[
            pltpu.VMEM((hist, per_b), jnp.int32),
            pltpu.VMEM((_NBUF, _CHUNK, dim), jnp.float32),
            pltpu.VMEM((_NBUF, _CHUNK, dim), jnp.float32),
            pltpu.SemaphoreType.DMA,
            pltpu.SemaphoreType.DMA((_NBUF,)),
            pltpu.SemaphoreType.DMA((_NBUF,)),
        ],
        compiler_params=pltpu.CompilerParams(use_tc_tiling_on_sc=False),
    )
    def emb_kernel(table_hbm, idx_hbm, out_hbm, idx_all, bin_v, bout_v,
                   sem_idx, sem_in, sem_out):
        wid = lax.axis_index("s") * nc + lax.axis_index("c")
        wb0 = wid * per_b

        def fire(h, c):
            pltpu.async_copy(idx_hbm.at[pl.ds(h * batch + wb0, per_b)],
                             idx_all.at[h], sem_idx)
            return c

        lax.fori_loop(0, hist, fire, 0)

        def drain(h, c):
            pltpu.make_async_copy(idx_hbm.at[pl.ds(0, per_b)],
                                  idx_all.at[0], sem_idx).wait()
            return c

        lax.fori_loop(0, hist, drain, 0)

        def start_gather(h, j, b):
            pltpu.async_copy(
                table_hbm.at[idx_all.at[h, pl.ds(j * _CHUNK, _CHUNK)]],
                bin_v.at[b], sem_in.at[b])

        def wait_gather(b):
            pltpu.make_async_copy(
                table_hbm.at[idx_all.at[0, pl.ds(0, _CHUNK)]], bin_v.at[b],
                sem_in.at[b]).wait()

        def start_wb(h, j, b):
            pltpu.async_copy(
                bout_v.at[b],
                out_hbm.at[h, pl.ds(wb0 + j * _CHUNK, _CHUNK)],
                sem_out.at[b])

        def wait_wb(b):
            pltpu.make_async_copy(bout_v.at[b],
                                  out_hbm.at[0, pl.ds(0, _CHUNK)],
                                  sem_out.at[b]).wait()

        def compute(b):
            def sbody(i, c):
                for dr in range(_ROWS_PER_IT):
                    r = i * _ROWS_PER_IT + dr
                    for c4 in range(dim // _LANES):
                        sl = pl.ds(c4 * _LANES, _LANES)
                        bout_v[b, r, sl] = bin_v[b, r, sl] * scale
                return c

            lax.fori_loop(0, _CHUNK // _ROWS_PER_IT, sbody, 0)

        for b in range(_NBUF):
            start_gather(0, b, b)

        def mbody(h, c):
            for b in range(_NBUF):
                wait_gather(b)

                @pl.when(h >= 1)
                def _():
                    wait_wb(b)

                compute(b)

                @pl.when(h + 1 < hist)
                def _():
                    start_gather(h + 1, b, b)

                start_wb(h, b, b)
            return c

        lax.fori_loop(0, hist, mbody, 0)
        for b in range(_NBUF):
            wait_wb(b)

    return emb_kernel


def kernel(persona, table):
    batch, hist = persona.shape
    vocab, dim = table.shape
    idx_flat = jnp.ravel(persona.T).astype(jnp.int32)
    out3 = _build(vocab, dim, batch, hist)(table, idx_flat)
    return jnp.transpose(out3, (1, 0, 2))
